# baseline (speedup 1.0000x reference)
import functools

import jax
import jax.numpy as jnp
from jax import lax
from jax.experimental import pallas as pl
from jax.experimental.pallas import tpu as pltpu
from jax.experimental.pallas import tpu_sc as plsc

N = 16777216
KK = 64
LANES = 16
NUM_WORKERS = 32
PER_WORKER = N // NUM_WORKERS
CHUNK = 16384
NCHUNKS = PER_WORKER // CHUNK
UNROLL = 8
INNER = CHUNK // (LANES * UNROLL)


def _sc_body(h_hbm, thr_hbm, out_hbm, thr_v,
             in_v0, in_v1, out_v0, out_v1,
             isem0, isem1, osem0, osem1):
    wid = lax.axis_index("s") * 2 + lax.axis_index("c")
    pltpu.sync_copy(thr_hbm, thr_v)
    in_bufs = (in_v0, in_v1)
    out_bufs = (out_v0, out_v1)
    isems = (isem0, isem1)
    osems = (osem0, osem1)
    wbase = wid * PER_WORKER

    def splat(j):
        return thr_v[pl.ds(j * LANES, LANES)]

    r31, r15, r47 = splat(31), splat(15), splat(47)
    r7, r23, r39, r55 = splat(7), splat(23), splat(39), splat(55)
    r3, r11, r19, r27 = splat(3), splat(11), splat(19), splat(27)
    r35, r43, r51, r59 = splat(35), splat(43), splat(51), splat(59)
    lane = jax.lax.iota(jnp.int32, LANES)

    def in_copy(g, b):
        return pltpu.make_async_copy(
            h_hbm.at[pl.ds(wbase + g * CHUNK, CHUNK)], in_bufs[b], isems[b])

    def out_copy(g, b):
        return pltpu.make_async_copy(
            out_bufs[b], out_hbm.at[pl.ds(wbase + g * CHUNK, CHUNK)], osems[b])

    in_copy(0, 0).start()
    in_copy(1, 1).start()

    def compute(in_v, out_v):
        @plsc.parallel_loop(0, INNER, unroll=1)
        def inner(i):
            offs = i * (LANES * UNROLL)
            for u in range(UNROLL):
                v = in_v[pl.ds(offs + u * LANES, LANES)]
                c1 = r31 < v
                pos = jnp.where(c1, 32 * LANES, 0) | lane
                t = jnp.where(c1, r47, r15)
                c2 = t < v
                pos = pos | jnp.where(c2, 16 * LANES, 0)
                t = jnp.where(c2, jnp.where(c1, r55, r23),
                              jnp.where(c1, r39, r7))
                c3 = t < v
                pos = pos | jnp.where(c3, 8 * LANES, 0)
                hi = jnp.where(c1, jnp.where(c2, jnp.where(c3, r59, r51),
                                             jnp.where(c3, r43, r35)),
                               jnp.where(c2, jnp.where(c3, r27, r19),
                                         jnp.where(c3, r11, r3)))
                pos = pos | jnp.where(hi < v, 4 * LANES, 0)
                for step in (2, 1):
                    t = plsc.load_gather(thr_v, [pos | ((step - 1) * LANES)])
                    pos = pos | jnp.where(t < v, step * LANES, 0)
                t = plsc.load_gather(thr_v, [pos])
                cnt = (pos >> 4) + jnp.where(t < v, 1, 0)
                out_v[pl.ds(offs + u * LANES, LANES)] = (
                    cnt.astype(jnp.float32) * (1.0 / KK))

    def pair_body(p, _):
        for b in (0, 1):
            g = 2 * p + b
            in_copy(g, b).wait()

            @pl.when(g >= 2)
            def _():
                out_copy(g - 2, b).wait()

            compute(in_bufs[b], out_bufs[b])
            out_copy(g, b).start()

            @pl.when(g + 2 < NCHUNKS)
            def _():
                in_copy(g + 2, b).start()

        return 0

    lax.fori_loop(0, NCHUNKS // 2, pair_body, 0)
    out_copy(NCHUNKS - 2, 0).wait()
    out_copy(NCHUNKS - 1, 1).wait()


@jax.jit
def _spray_bank_sc(h_scaled, thr_row):
    f = pl.kernel(
        _sc_body,
        out_type=jax.ShapeDtypeStruct((N,), jnp.float32),
        mesh=plsc.VectorSubcoreMesh(core_axis_name="c", subcore_axis_name="s"),
        scratch_types=[
            pltpu.VMEM((KK * LANES,), jnp.float32),
            pltpu.VMEM((CHUNK,), jnp.float32),
            pltpu.VMEM((CHUNK,), jnp.float32),
            pltpu.VMEM((CHUNK,), jnp.float32),
            pltpu.VMEM((CHUNK,), jnp.float32),
            pltpu.SemaphoreType.DMA,
            pltpu.SemaphoreType.DMA,
            pltpu.SemaphoreType.DMA,
            pltpu.SemaphoreType.DMA,
        ],
        compiler_params=pltpu.CompilerParams(needs_layout_passes=False),
    )
    return f(h_scaled, thr_row)


def kernel(h_scaled, thresholds, bank_idx):
    row = jnp.take(thresholds, bank_idx % thresholds.shape[0], axis=0)
    rep = jnp.reshape(jnp.broadcast_to(row[:, None], (KK, LANES)), (KK * LANES,))
    return _spray_bank_sc(h_scaled, rep)


# speedup vs baseline: 1.1034x; 1.1034x over previous
import functools

import jax
import jax.numpy as jnp
from jax import lax
from jax.experimental import pallas as pl
from jax.experimental.pallas import tpu as pltpu
from jax.experimental.pallas import tpu_sc as plsc

N = 16777216
KK = 64
LANES = 16
NUM_WORKERS = 32
PER_WORKER = N // NUM_WORKERS
CHUNK = 16384
NCHUNKS = PER_WORKER // CHUNK
UNROLL = 4
INNER = CHUNK // (LANES * UNROLL)


def _sc_body(h_hbm, thr_hbm, out_hbm, thr_v,
             in_v0, in_v1, out_v0, out_v1,
             isem0, isem1, osem0, osem1):
    wid = lax.axis_index("s") * 2 + lax.axis_index("c")
    pltpu.sync_copy(thr_hbm, thr_v)
    in_bufs = (in_v0, in_v1)
    out_bufs = (out_v0, out_v1)
    isems = (isem0, isem1)
    osems = (osem0, osem1)
    wbase = wid * PER_WORKER

    def splat(j):
        return thr_v[pl.ds(j * LANES, LANES)]

    r31, r15, r47 = splat(31), splat(15), splat(47)
    r7, r23, r39, r55 = splat(7), splat(23), splat(39), splat(55)
    r3, r11, r19, r27 = splat(3), splat(11), splat(19), splat(27)
    r35, r43, r51, r59 = splat(35), splat(43), splat(51), splat(59)
    lane = jax.lax.iota(jnp.int32, LANES)

    def in_copy(g, b):
        return pltpu.make_async_copy(
            h_hbm.at[pl.ds(wbase + g * CHUNK, CHUNK)], in_bufs[b], isems[b])

    def out_copy(g, b):
        return pltpu.make_async_copy(
            out_bufs[b], out_hbm.at[pl.ds(wbase + g * CHUNK, CHUNK)], osems[b])

    in_copy(0, 0).start()
    in_copy(1, 1).start()

    def compute(in_v, out_v):
        @plsc.parallel_loop(0, INNER, unroll=1)
        def inner(i):
            offs = i * (LANES * UNROLL)
            for u in range(UNROLL):
                v = in_v[pl.ds(offs + u * LANES, LANES)]
                c1 = r31 < v
                pos = jnp.where(c1, 32 * LANES, 0) | lane
                t = jnp.where(c1, r47, r15)
                c2 = t < v
                pos = pos | jnp.where(c2, 16 * LANES, 0)
                t = jnp.where(c2, jnp.where(c1, r55, r23),
                              jnp.where(c1, r39, r7))
                c3 = t < v
                pos = pos | jnp.where(c3, 8 * LANES, 0)
                for step in (4, 2, 1):
                    t = plsc.load_gather(thr_v, [pos | ((step - 1) * LANES)])
                    pos = pos | jnp.where(t < v, step * LANES, 0)
                t = plsc.load_gather(thr_v, [pos])
                cnt = (pos >> 4) + jnp.where(t < v, 1, 0)
                out_v[pl.ds(offs + u * LANES, LANES)] = (
                    cnt.astype(jnp.float32) * (1.0 / KK))

    def pair_body(p, _):
        for b in (0, 1):
            g = 2 * p + b
            in_copy(g, b).wait()

            @pl.when(g >= 2)
            def _():
                out_copy(g - 2, b).wait()

            compute(in_bufs[b], out_bufs[b])
            out_copy(g, b).start()

            @pl.when(g + 2 < NCHUNKS)
            def _():
                in_copy(g + 2, b).start()

        return 0

    lax.fori_loop(0, NCHUNKS // 2, pair_body, 0)
    out_copy(NCHUNKS - 2, 0).wait()
    out_copy(NCHUNKS - 1, 1).wait()


@jax.jit
def _spray_bank_sc(h_scaled, thr_row):
    f = pl.kernel(
        _sc_body,
        out_type=jax.ShapeDtypeStruct((N,), jnp.float32),
        mesh=plsc.VectorSubcoreMesh(core_axis_name="c", subcore_axis_name="s"),
        scratch_types=[
            pltpu.VMEM((KK * LANES,), jnp.float32),
            pltpu.VMEM((CHUNK,), jnp.float32),
            pltpu.VMEM((CHUNK,), jnp.float32),
            pltpu.VMEM((CHUNK,), jnp.float32),
            pltpu.VMEM((CHUNK,), jnp.float32),
            pltpu.SemaphoreType.DMA,
            pltpu.SemaphoreType.DMA,
            pltpu.SemaphoreType.DMA,
            pltpu.SemaphoreType.DMA,
        ],
        compiler_params=pltpu.CompilerParams(needs_layout_passes=False),
    )
    return f(h_scaled, thr_row)


def kernel(h_scaled, thresholds, bank_idx):
    row = jnp.take(thresholds, bank_idx % thresholds.shape[0], axis=0)
    rep = jnp.reshape(jnp.broadcast_to(row[:, None], (KK, LANES)), (KK * LANES,))
    return _spray_bank_sc(h_scaled, rep)


# speedup vs baseline: 1.1195x; 1.0146x over previous
import functools

import jax
import jax.numpy as jnp
from jax import lax
from jax.experimental import pallas as pl
from jax.experimental.pallas import tpu as pltpu
from jax.experimental.pallas import tpu_sc as plsc

N = 16777216
KK = 64
LANES = 16
NUM_WORKERS = 32
PER_WORKER = N // NUM_WORKERS
CHUNK = 16384
NCHUNKS = PER_WORKER // CHUNK
UNROLL = 4
INNER = CHUNK // (LANES * UNROLL)


def _sc_body(h_hbm, thr_hbm, out_hbm, thr_v,
             in_v0, in_v1, out_v0, out_v1,
             isem0, isem1, osem0, osem1):
    wid = lax.axis_index("s") * 2 + lax.axis_index("c")
    pltpu.sync_copy(thr_hbm, thr_v)
    in_bufs = (in_v0, in_v1)
    out_bufs = (out_v0, out_v1)
    isems = (isem0, isem1)
    osems = (osem0, osem1)
    wbase = wid * PER_WORKER

    def splat(j):
        return thr_v[pl.ds(j * LANES, LANES)]

    r31, r15, r47 = splat(31), splat(15), splat(47)
    r7, r23, r39, r55 = splat(7), splat(23), splat(39), splat(55)
    r3, r11, r19, r27 = splat(3), splat(11), splat(19), splat(27)
    r35, r43, r51, r59 = splat(35), splat(43), splat(51), splat(59)
    lane = jax.lax.iota(jnp.int32, LANES)

    def in_copy(g, b):
        return pltpu.make_async_copy(
            h_hbm.at[pl.ds(wbase + g * CHUNK, CHUNK)], in_bufs[b], isems[b])

    def out_copy(g, b):
        return pltpu.make_async_copy(
            out_bufs[b], out_hbm.at[pl.ds(wbase + g * CHUNK, CHUNK)], osems[b])

    in_copy(0, 0).start()
    in_copy(1, 1).start()

    def compute(in_v, out_v):
        @plsc.parallel_loop(0, INNER, unroll=1)
        def inner(i):
            offs = i * (LANES * UNROLL)
            for u in range(UNROLL):
                v = in_v[pl.ds(offs + u * LANES, LANES)]
                c1 = r31 < v
                pos = jnp.where(c1, 32 * LANES, 0) | lane
                t = jnp.where(c1, r47, r15)
                c2 = t < v
                pos = pos | jnp.where(c2, 16 * LANES, 0)
                for step in (8, 4, 2, 1):
                    t = plsc.load_gather(thr_v, [pos | ((step - 1) * LANES)])
                    pos = pos | jnp.where(t < v, step * LANES, 0)
                t = plsc.load_gather(thr_v, [pos])
                cnt = (pos >> 4) + jnp.where(t < v, 1, 0)
                out_v[pl.ds(offs + u * LANES, LANES)] = (
                    cnt.astype(jnp.float32) * (1.0 / KK))

    def pair_body(p, _):
        for b in (0, 1):
            g = 2 * p + b
            in_copy(g, b).wait()

            @pl.when(g >= 2)
            def _():
                out_copy(g - 2, b).wait()

            compute(in_bufs[b], out_bufs[b])
            out_copy(g, b).start()

            @pl.when(g + 2 < NCHUNKS)
            def _():
                in_copy(g + 2, b).start()

        return 0

    lax.fori_loop(0, NCHUNKS // 2, pair_body, 0)
    out_copy(NCHUNKS - 2, 0).wait()
    out_copy(NCHUNKS - 1, 1).wait()


@jax.jit
def _spray_bank_sc(h_scaled, thr_row):
    f = pl.kernel(
        _sc_body,
        out_type=jax.ShapeDtypeStruct((N,), jnp.float32),
        mesh=plsc.VectorSubcoreMesh(core_axis_name="c", subcore_axis_name="s"),
        scratch_types=[
            pltpu.VMEM((KK * LANES,), jnp.float32),
            pltpu.VMEM((CHUNK,), jnp.float32),
            pltpu.VMEM((CHUNK,), jnp.float32),
            pltpu.VMEM((CHUNK,), jnp.float32),
            pltpu.VMEM((CHUNK,), jnp.float32),
            pltpu.SemaphoreType.DMA,
            pltpu.SemaphoreType.DMA,
            pltpu.SemaphoreType.DMA,
            pltpu.SemaphoreType.DMA,
        ],
        compiler_params=pltpu.CompilerParams(needs_layout_passes=False),
    )
    return f(h_scaled, thr_row)


def kernel(h_scaled, thresholds, bank_idx):
    row = jnp.take(thresholds, bank_idx % thresholds.shape[0], axis=0)
    rep = jnp.reshape(jnp.broadcast_to(row[:, None], (KK, LANES)), (KK * LANES,))
    return _spray_bank_sc(h_scaled, rep)
